# Initial kernel scaffold; baseline (speedup 1.0000x reference)
import functools

import jax
import jax.numpy as jnp
from jax import lax
from jax.experimental import pallas as pl
from jax.experimental.pallas import tpu as pltpu
from jax.experimental.pallas import tpu_sc as plsc

N = 10000
E = 320000
D = 128

NC = 2
NS = 16
NW = NC * NS

B = 128
CH = -(-E // (NW * B))
E_PAD = NW * CH * B

N_PAD = 10016
ROWS_PER_TILE = N_PAD // NS


def _sc_aggregate_body(x_hbm, src_hbm, dst_hbm, zeros_hbm, out_hbm,
                       src_v, dst_v, rows_a, rows_b, sem_a, sem_b):
    cid = lax.axis_index("c")
    sid = lax.axis_index("s")
    wid = cid * NS + sid

    row0 = sid * ROWS_PER_TILE

    def zero_and_run(acc):
        pltpu.sync_copy(zeros_hbm, acc.at[pl.ds(row0, ROWS_PER_TILE)])
        pltpu.sync_copy(src_hbm.at[wid], src_v)
        pltpu.sync_copy(dst_hbm.at[wid], dst_v)
        plsc.subcore_barrier()

        pltpu.async_copy(x_hbm.at[src_v.at[0]], rows_a, sem_a)

        def step(j, _):
            even = (j % 2) == 0

            @pl.when(j + 1 < CH)
            def _prefetch():
                @pl.when(even)
                def _():
                    pltpu.async_copy(x_hbm.at[src_v.at[j + 1]], rows_b, sem_b)

                @pl.when(jnp.logical_not(even))
                def _():
                    pltpu.async_copy(x_hbm.at[src_v.at[j + 1]], rows_a, sem_a)

            @pl.when(even)
            def _scatter_even():
                pltpu.make_async_copy(x_hbm.at[src_v.at[j]], rows_a, sem_a).wait()
                pltpu.sync_copy(rows_a, acc.at[dst_v.at[j]], add=True)

            @pl.when(jnp.logical_not(even))
            def _scatter_odd():
                pltpu.make_async_copy(x_hbm.at[src_v.at[j]], rows_b, sem_b).wait()
                pltpu.sync_copy(rows_b, acc.at[dst_v.at[j]], add=True)

            return ()

        lax.fori_loop(0, CH, step, (), unroll=False)

        plsc.subcore_barrier()
        pltpu.sync_copy(acc.at[pl.ds(row0, ROWS_PER_TILE)],
                        out_hbm.at[cid, pl.ds(row0, ROWS_PER_TILE)])

    pl.run_scoped(
        zero_and_run,
        plsc.MemoryRef((N_PAD, D), jnp.float32, pltpu.VMEM_SHARED),
    )


@jax.jit
def _sc_aggregate(x, src_p, dst_p, zeros_tile):
    mesh = plsc.VectorSubcoreMesh(core_axis_name="c", subcore_axis_name="s")
    return pl.kernel(
        _sc_aggregate_body,
        out_type=jax.ShapeDtypeStruct((NC, N_PAD, D), jnp.float32),
        mesh=mesh,
        scratch_types=[
            pltpu.VMEM((CH, B), jnp.int32),
            pltpu.VMEM((CH, B), jnp.int32),
            pltpu.VMEM((B, D), jnp.float32),
            pltpu.VMEM((B, D), jnp.float32),
            pltpu.SemaphoreType.DMA,
            pltpu.SemaphoreType.DMA,
        ],
    )(x, src_p, dst_p, zeros_tile)


def _tc_combine_body(p_ref, w_ref, b_ref, o_ref):
    s = p_ref[0] + p_ref[1]
    y = jnp.dot(s, w_ref[...], preferred_element_type=jnp.float32)
    o_ref[...] = jnp.maximum(y + b_ref[...], 0.0)


BM = 2504


@jax.jit
def _tc_combine(partials, weight, bias2d):
    return pl.pallas_call(
        _tc_combine_body,
        grid=(N_PAD // BM,),
        in_specs=[
            pl.BlockSpec((NC, BM, D), lambda i: (0, i, 0)),
            pl.BlockSpec((D, D), lambda i: (0, 0)),
            pl.BlockSpec((1, D), lambda i: (0, 0)),
        ],
        out_specs=pl.BlockSpec((BM, D), lambda i: (i, 0)),
        out_shape=jax.ShapeDtypeStruct((N_PAD, D), jnp.float32),
    )(partials, weight, bias2d)


def kernel(adj, x, weight, bias):
    dst = adj[0]
    src = adj[1]
    pad = E_PAD - E
    src_p = jnp.concatenate([src, jnp.zeros((pad,), jnp.int32)]).reshape(NW, CH, B)
    dst_p = jnp.concatenate([dst, jnp.full((pad,), N, jnp.int32)]).reshape(NW, CH, B)
    zeros_tile = jnp.zeros((ROWS_PER_TILE, D), jnp.float32)

    partials = _sc_aggregate(x, src_p, dst_p, zeros_tile)
    out = _tc_combine(partials, weight, bias.reshape(1, D))
    return out[:N]


# speedup vs baseline: 3.4820x; 3.4820x over previous
import functools

import jax
import jax.numpy as jnp
from jax import lax
from jax.experimental import pallas as pl
from jax.experimental.pallas import tpu as pltpu
from jax.experimental.pallas import tpu_sc as plsc

N = 10000
E = 320000
D = 128

NC = 2
NS = 16
NW = NC * NS

B = 128
CH = 80
E_PAD = NW * CH * B
G = 16
NG = CH // G

N_PAD = 10112
ROWS_PER_TILE = N_PAD // NS


def _sc_aggregate_body(x_hbm, src_hbm, dst_hbm, zeros_hbm, out_hbm,
                       src_v, dst_v, rows_a, rows_b, acc, sem_a, sem_b):
    cid = lax.axis_index("c")
    sid = lax.axis_index("s")
    wid = cid * NS + sid

    row0 = sid * ROWS_PER_TILE

    pltpu.sync_copy(zeros_hbm, acc.at[pl.ds(row0, ROWS_PER_TILE)])
    plsc.subcore_barrier()

    rows = (rows_a, rows_b)
    sems = (sem_a, sem_b)

    def group(g, _):
        pltpu.sync_copy(src_hbm.at[wid, pl.ds(g * G, G)], src_v)
        pltpu.sync_copy(dst_hbm.at[wid, pl.ds(g * G, G)], dst_v)
        pltpu.async_copy(x_hbm.at[src_v.at[0]], rows[0], sems[0])
        for j in range(G):
            if j + 1 < G:
                pltpu.async_copy(x_hbm.at[src_v.at[j + 1]],
                                 rows[(j + 1) % 2], sems[(j + 1) % 2])
            pltpu.make_async_copy(x_hbm.at[src_v.at[j]],
                                  rows[j % 2], sems[j % 2]).wait()
            pltpu.sync_copy(rows[j % 2], acc.at[dst_v.at[j]], add=True)
        return ()

    lax.fori_loop(0, NG, group, (), unroll=False)

    plsc.subcore_barrier()
    pltpu.sync_copy(acc.at[pl.ds(row0, ROWS_PER_TILE)],
                    out_hbm.at[cid, pl.ds(row0, ROWS_PER_TILE)])


@jax.jit
def _sc_aggregate(x, src_p, dst_p, zeros_tile):
    mesh = plsc.VectorSubcoreMesh(core_axis_name="c", subcore_axis_name="s")
    return pl.kernel(
        _sc_aggregate_body,
        out_type=jax.ShapeDtypeStruct((NC, N_PAD, D), jnp.float32),
        mesh=mesh,
        scratch_types=[
            pltpu.VMEM((G, B), jnp.int32),
            pltpu.VMEM((G, B), jnp.int32),
            pltpu.VMEM((B, D), jnp.float32),
            pltpu.VMEM((B, D), jnp.float32),
            pltpu.VMEM_SHARED((N_PAD, D), jnp.float32),
            pltpu.SemaphoreType.DMA,
            pltpu.SemaphoreType.DMA,
        ],
    )(x, src_p, dst_p, zeros_tile)


def _tc_combine_body(p_ref, w_ref, b_ref, o_ref):
    s = p_ref[0] + p_ref[1]
    y = jnp.dot(s, w_ref[...], preferred_element_type=jnp.float32,
                precision=jax.lax.Precision.HIGHEST)
    o_ref[...] = jnp.maximum(y + b_ref[...], 0.0)


BM = N_PAD // 4


@jax.jit
def _tc_combine(partials, weight, bias2d):
    return pl.pallas_call(
        _tc_combine_body,
        grid=(N_PAD // BM,),
        in_specs=[
            pl.BlockSpec((NC, BM, D), lambda i: (0, i, 0)),
            pl.BlockSpec((D, D), lambda i: (0, 0)),
            pl.BlockSpec((1, D), lambda i: (0, 0)),
        ],
        out_specs=pl.BlockSpec((BM, D), lambda i: (i, 0)),
        out_shape=jax.ShapeDtypeStruct((N_PAD, D), jnp.float32),
    )(partials, weight, bias2d)


def kernel(adj, x, weight, bias):
    dst = adj[0]
    src = adj[1]
    pad = E_PAD - E
    src_p = jnp.concatenate([src, jnp.zeros((pad,), jnp.int32)]).reshape(NW, CH, B)
    dst_p = jnp.concatenate([dst, jnp.full((pad,), N, jnp.int32)]).reshape(NW, CH, B)
    zeros_tile = jnp.zeros((ROWS_PER_TILE, D), jnp.float32)

    partials = _sc_aggregate(x, src_p, dst_p, zeros_tile)
    out = _tc_combine(partials, weight, bias.reshape(1, D))
    return out[:N]


# speedup vs baseline: 11.6219x; 3.3377x over previous
import functools

import jax
import jax.numpy as jnp
from jax import lax
from jax.experimental import pallas as pl
from jax.experimental.pallas import tpu as pltpu
from jax.experimental.pallas import tpu_sc as plsc

N = 10000
E = 320000
D = 128

NC = 2
NS = 16
NW = NC * NS

B = 128
CH = 80
E_PAD = NW * CH * B
G = 16
NG = CH // G

N_PAD = 10112
ROWS_PER_TILE = N_PAD // NS


def _sc_aggregate_body(x_hbm, src_hbm, dst_hbm, zeros_hbm, out_hbm,
                       src_v, dst_v, rows_a, rows_b, acc, sem_a, sem_b):
    cid = lax.axis_index("c")
    sid = lax.axis_index("s")
    wid = cid * NS + sid

    row0 = sid * ROWS_PER_TILE

    pltpu.sync_copy(zeros_hbm.at[pl.ds(row0, ROWS_PER_TILE)],
                    acc.at[pl.ds(row0, ROWS_PER_TILE)])
    plsc.subcore_barrier()

    rows = (rows_a, rows_b)
    sems = (sem_a, sem_b)

    def group(g, _):
        pltpu.sync_copy(src_hbm.at[wid, pl.ds(g * G, G)], src_v)
        pltpu.sync_copy(dst_hbm.at[wid, pl.ds(g * G, G)], dst_v)
        pltpu.async_copy(x_hbm.at[src_v.at[0]], rows[0], sems[0])
        for j in range(G):
            if j + 1 < G:
                pltpu.async_copy(x_hbm.at[src_v.at[j + 1]],
                                 rows[(j + 1) % 2], sems[(j + 1) % 2])
            pltpu.make_async_copy(x_hbm.at[src_v.at[j]],
                                  rows[j % 2], sems[j % 2]).wait()
            pltpu.sync_copy(rows[j % 2], acc.at[dst_v.at[j]], add=True)
        return ()

    lax.fori_loop(0, NG, group, (), unroll=False)

    plsc.subcore_barrier()
    pltpu.sync_copy(acc.at[pl.ds(row0, ROWS_PER_TILE)],
                    out_hbm.at[cid, pl.ds(row0, ROWS_PER_TILE)])


@jax.jit
def _sc_aggregate(x, src_p, dst_p, zeros_tile):
    mesh = plsc.VectorSubcoreMesh(core_axis_name="c", subcore_axis_name="s")
    return pl.kernel(
        _sc_aggregate_body,
        out_type=jax.ShapeDtypeStruct((NC, N_PAD, D), jnp.float32),
        mesh=mesh,
        scratch_types=[
            pltpu.VMEM((G, B), jnp.int32),
            pltpu.VMEM((G, B), jnp.int32),
            pltpu.VMEM((B, D), jnp.float32),
            pltpu.VMEM((B, D), jnp.float32),
            pltpu.VMEM_SHARED((N_PAD, D), jnp.float32),
            pltpu.SemaphoreType.DMA,
            pltpu.SemaphoreType.DMA,
        ],
    )(x, src_p, dst_p, zeros_tile)


def _tc_combine_body(p_ref, w_ref, b_ref, o_ref):
    s = p_ref[0] + p_ref[1]
    y = jnp.dot(s, w_ref[...], preferred_element_type=jnp.float32,
                precision=jax.lax.Precision.HIGHEST)
    o_ref[...] = jnp.maximum(y + b_ref[...], 0.0)


BM = N_PAD // 4


@jax.jit
def _tc_combine(partials, weight, bias2d):
    return pl.pallas_call(
        _tc_combine_body,
        grid=(N_PAD // BM,),
        in_specs=[
            pl.BlockSpec((NC, BM, D), lambda i: (0, i, 0)),
            pl.BlockSpec((D, D), lambda i: (0, 0)),
            pl.BlockSpec((1, D), lambda i: (0, 0)),
        ],
        out_specs=pl.BlockSpec((BM, D), lambda i: (i, 0)),
        out_shape=jax.ShapeDtypeStruct((N_PAD, D), jnp.float32),
    )(partials, weight, bias2d)


def kernel(adj, x, weight, bias):
    dst = adj[0]
    src = adj[1]
    pad = E_PAD - E
    pad_iota = jnp.arange(pad, dtype=jnp.int32)
    src_p = jnp.concatenate([src, pad_iota % N]).reshape(NW, CH, B)
    dst_p = jnp.concatenate([dst, N + pad_iota % (N_PAD - N)]).reshape(NW, CH, B)
    zeros_full = jnp.zeros((N_PAD, D), jnp.float32)

    partials = _sc_aggregate(x, src_p, dst_p, zeros_full)
    out = _tc_combine(partials, weight, bias.reshape(1, D))
    return out[:N]


# speedup vs baseline: 12.4548x; 1.0717x over previous
import functools

import jax
import jax.numpy as jnp
from jax import lax
from jax.experimental import pallas as pl
from jax.experimental.pallas import tpu as pltpu
from jax.experimental.pallas import tpu_sc as plsc

N = 10000
E = 320000
D = 128

NC = 2
NS = 16
NW = NC * NS

B = 128
CH = 80
E_PAD = NW * CH * B
G = 16
NG = CH // G

N_PAD = 10112
ROWS_PER_TILE = N_PAD // NS


def _sc_aggregate_body(x_hbm, src_hbm, dst_hbm, zeros_hbm, out_hbm,
                       src_v, dst_v, rows_a, rows_b, acc,
                       gsem_a, gsem_b, ssem_a, ssem_b, isem_a, isem_b):
    cid = lax.axis_index("c")
    sid = lax.axis_index("s")
    wid = cid * NS + sid

    row0 = sid * ROWS_PER_TILE

    pltpu.sync_copy(zeros_hbm.at[pl.ds(row0, ROWS_PER_TILE)],
                    acc.at[pl.ds(row0, ROWS_PER_TILE)])
    plsc.subcore_barrier()

    rows = (rows_a, rows_b)
    gsems = (gsem_a, gsem_b)
    ssems = (ssem_a, ssem_b)

    def stage_idx(g, slot):
        pltpu.async_copy(src_hbm.at[wid, pl.ds(g * G, G)], src_v.at[slot],
                         isem_a)
        pltpu.async_copy(dst_hbm.at[wid, pl.ds(g * G, G)], dst_v.at[slot],
                         isem_b)

    def wait_idx(g, slot):
        pltpu.make_async_copy(src_hbm.at[wid, pl.ds(g * G, G)],
                              src_v.at[slot], isem_a).wait()
        pltpu.make_async_copy(dst_hbm.at[wid, pl.ds(g * G, G)],
                              dst_v.at[slot], isem_b).wait()

    def wait_gather(buf):
        pltpu.make_async_copy(x_hbm.at[src_v.at[0, 0]], rows[buf],
                              gsems[buf]).wait()

    def wait_scatter(buf):
        pltpu.make_async_copy(rows[buf], acc.at[dst_v.at[0, 0]],
                              ssems[buf]).wait()

    stage_idx(0, 0)
    wait_idx(0, 0)
    pltpu.async_copy(x_hbm.at[src_v.at[0, 0]], rows[0], gsems[0])

    def group(g, _):
        slot = g % 2

        @pl.when(g + 1 < NG)
        def _stage_next():
            stage_idx(g + 1, (g + 1) % 2)

        for j in range(G):
            jj = g * G + j
            par = (j + 1) % 2
            if j + 1 < G:
                @pl.when((jj >= 1) & (jj + 1 < CH))
                def _drain():
                    wait_scatter(par)

                @pl.when(jj + 1 < CH)
                def _prefetch():
                    pltpu.async_copy(x_hbm.at[src_v.at[slot, j + 1]],
                                     rows[par], gsems[par])
            else:
                @pl.when(g + 1 < NG)
                def _prefetch_group():
                    wait_scatter(par)
                    wait_idx(g + 1, (g + 1) % 2)
                    pltpu.async_copy(x_hbm.at[src_v.at[(g + 1) % 2, 0]],
                                     rows[par], gsems[par])
            wait_gather(j % 2)
            pltpu.async_copy(rows[j % 2], acc.at[dst_v.at[slot, j]],
                             ssems[j % 2], add=True)
        return ()

    lax.fori_loop(0, NG, group, (), unroll=False)

    wait_scatter((CH - 2) % 2)
    wait_scatter((CH - 1) % 2)

    plsc.subcore_barrier()
    pltpu.sync_copy(acc.at[pl.ds(row0, ROWS_PER_TILE)],
                    out_hbm.at[cid, pl.ds(row0, ROWS_PER_TILE)])


@jax.jit
def _sc_aggregate(x, src_p, dst_p, zeros_tile):
    mesh = plsc.VectorSubcoreMesh(core_axis_name="c", subcore_axis_name="s")
    return pl.kernel(
        _sc_aggregate_body,
        out_type=jax.ShapeDtypeStruct((NC, N_PAD, D), jnp.float32),
        mesh=mesh,
        scratch_types=[
            pltpu.VMEM((2, G, B), jnp.int32),
            pltpu.VMEM((2, G, B), jnp.int32),
            pltpu.VMEM((B, D), jnp.float32),
            pltpu.VMEM((B, D), jnp.float32),
            pltpu.VMEM_SHARED((N_PAD, D), jnp.float32),
            pltpu.SemaphoreType.DMA,
            pltpu.SemaphoreType.DMA,
            pltpu.SemaphoreType.DMA,
            pltpu.SemaphoreType.DMA,
            pltpu.SemaphoreType.DMA,
            pltpu.SemaphoreType.DMA,
        ],
    )(x, src_p, dst_p, zeros_tile)


def _tc_combine_body(p_ref, w_ref, b_ref, o_ref):
    s = p_ref[0] + p_ref[1]
    y = jnp.dot(s, w_ref[...], preferred_element_type=jnp.float32,
                precision=jax.lax.Precision.HIGHEST)
    o_ref[...] = jnp.maximum(y + b_ref[...], 0.0)


BM = N_PAD // 4


@jax.jit
def _tc_combine(partials, weight, bias2d):
    return pl.pallas_call(
        _tc_combine_body,
        grid=(N_PAD // BM,),
        in_specs=[
            pl.BlockSpec((NC, BM, D), lambda i: (0, i, 0)),
            pl.BlockSpec((D, D), lambda i: (0, 0)),
            pl.BlockSpec((1, D), lambda i: (0, 0)),
        ],
        out_specs=pl.BlockSpec((BM, D), lambda i: (i, 0)),
        out_shape=jax.ShapeDtypeStruct((N_PAD, D), jnp.float32),
    )(partials, weight, bias2d)


def kernel(adj, x, weight, bias):
    dst = adj[0]
    src = adj[1]
    pad = E_PAD - E
    pad_iota = jnp.arange(pad, dtype=jnp.int32)
    src_p = jnp.concatenate([src, pad_iota % N]).reshape(NW, CH, B)
    dst_p = jnp.concatenate([dst, N + pad_iota % (N_PAD - N)]).reshape(NW, CH, B)
    zeros_full = jnp.zeros((N_PAD, D), jnp.float32)

    partials = _sc_aggregate(x, src_p, dst_p, zeros_full)
    out = _tc_combine(partials, weight, bias.reshape(1, D))
    return out[:N]
